# Initial kernel scaffold; baseline (speedup 1.0000x reference)
import jax
import jax.numpy as jnp
from jax.experimental import pallas as pl


def kernel(x, edge_index, weights, biases):
    raise NotImplementedError("write your pallas kernel here")


# speedup vs baseline: 134.4587x; 134.4587x over previous
import functools

import jax
import jax.numpy as jnp
from jax import lax
from jax.experimental import pallas as pl
from jax.experimental.pallas import tpu as pltpu
from jax.experimental.pallas import tpu_sc as plsc

N = 100000
E = 1600000
NIN = 512
NOUT = 512
LO = N - NOUT

_info = plsc.get_sparse_core_info()
NC = _info.num_cores
NS = _info.num_subcores
L = _info.num_lanes
NW = NC * NS

E_PER = E // NW
CHUNK = 10000
NCHUNK = E_PER // CHUNK
VECS = CHUNK // L

assert E_PER * NW == E and NCHUNK * CHUNK == E_PER and VECS * L == CHUNK

_mesh = plsc.VectorSubcoreMesh(core_axis_name="c", subcore_axis_name="s")


@functools.partial(
    pl.kernel,
    mesh=_mesh,
    compiler_params=pltpu.CompilerParams(needs_layout_passes=False),
    out_type=jax.ShapeDtypeStruct((NW * NOUT,), jnp.float32),
    scratch_types=[
        pltpu.VMEM((NIN,), jnp.float32),
        pltpu.VMEM((NOUT,), jnp.float32),
        pltpu.VMEM((CHUNK,), jnp.int32),
        pltpu.VMEM((CHUNK,), jnp.int32),
        pltpu.VMEM((CHUNK,), jnp.float32),
    ],
)
def _edge_scan(x_hbm, edge_hbm, w_hbm, out_hbm, x_v, acc_v, src_v, dst_v, w_v):
    wid = lax.axis_index("s") * NC + lax.axis_index("c")
    base = wid * E_PER

    pltpu.sync_copy(x_hbm, x_v)

    zero = jnp.zeros((L,), jnp.float32)

    def zbody(i, carry):
        acc_v[pl.ds(i * L, L)] = zero
        return carry

    lax.fori_loop(0, NOUT // L, zbody, 0)

    for c in range(NCHUNK):
        off = base + c * CHUNK
        pltpu.sync_copy(edge_hbm.at[pl.ds(off, CHUNK)], src_v)
        pltpu.sync_copy(edge_hbm.at[pl.ds(E + off, CHUNK)], dst_v)
        pltpu.sync_copy(w_hbm.at[pl.ds(off, CHUNK)], w_v)

        def body(i, carry):
            s = src_v[pl.ds(i * L, L)]
            d = dst_v[pl.ds(i * L, L)]
            w = w_v[pl.ds(i * L, L)]
            m = (s < NIN) & (d >= LO)
            xg = plsc.load_gather(x_v, [s & (NIN - 1)])
            val = w * xg
            plsc.addupdate_scatter(acc_v, [(d - LO) & (NOUT - 1)], val, mask=m)
            return carry

        lax.fori_loop(0, VECS, body, 0)

    pltpu.sync_copy(acc_v, out_hbm.at[pl.ds(wid * NOUT, NOUT)])


def _combine_body(p_ref, b_ref, o_ref):
    o_ref[...] = jnp.tanh(
        jnp.sum(p_ref[...], axis=0, keepdims=True) + b_ref[...]
    )


def kernel(x, edge_index, weights, biases):
    partials = _edge_scan(x, edge_index.reshape(2 * E), weights)
    out = pl.pallas_call(
        _combine_body,
        out_shape=jax.ShapeDtypeStruct((1, NOUT), jnp.float32),
    )(partials.reshape(NW, NOUT), biases[LO:].reshape(1, NOUT))
    return out.reshape(NOUT)


# speedup vs baseline: 156.8925x; 1.1668x over previous
import functools

import jax
import jax.numpy as jnp
from jax import lax
from jax.experimental import pallas as pl
from jax.experimental.pallas import tpu as pltpu
from jax.experimental.pallas import tpu_sc as plsc

N = 100000
E = 1600000
NIN = 512
NOUT = 512
LO = N - NOUT

_info = plsc.get_sparse_core_info()
NC = _info.num_cores
NS = _info.num_subcores
L = _info.num_lanes
NW = NC * NS

E_PER = E // NW
CHUNK = 10000
NCHUNK = E_PER // CHUNK
VECS = CHUNK // L

NBUF = 2
U = 5

assert E_PER * NW == E and NCHUNK * CHUNK == E_PER and VECS * L == CHUNK
assert VECS % U == 0

_mesh = plsc.VectorSubcoreMesh(core_axis_name="c", subcore_axis_name="s")


@functools.partial(
    pl.kernel,
    mesh=_mesh,
    compiler_params=pltpu.CompilerParams(needs_layout_passes=False),
    out_type=jax.ShapeDtypeStruct((NW * NOUT,), jnp.float32),
    scratch_types=[
        pltpu.VMEM((NIN,), jnp.float32),
        pltpu.VMEM((NOUT,), jnp.float32),
        pltpu.VMEM((CHUNK,), jnp.int32),
        pltpu.VMEM((CHUNK,), jnp.int32),
        pltpu.VMEM((CHUNK,), jnp.int32),
        pltpu.VMEM((CHUNK,), jnp.int32),
        pltpu.VMEM((CHUNK,), jnp.float32),
        pltpu.VMEM((CHUNK,), jnp.float32),
        pltpu.SemaphoreType.DMA,
        pltpu.SemaphoreType.DMA,
    ],
)
def _edge_scan(x_hbm, edge_hbm, w_hbm, out_hbm,
               x_v, acc_v, src0, src1, dst0, dst1, w0, w1, sem0, sem1):
    sems = (sem0, sem1)
    srcs = (src0, src1)
    dsts = (dst0, dst1)
    ws = (w0, w1)
    wid = lax.axis_index("s") * NC + lax.axis_index("c")
    base = wid * E_PER

    pltpu.sync_copy(x_hbm, x_v)

    zero = jnp.zeros((L,), jnp.float32)

    def zbody(i, carry):
        acc_v[pl.ds(i * L, L)] = zero
        return carry

    lax.fori_loop(0, NOUT // L, zbody, 0)

    def start(c):
        b = c % NBUF
        off = base + c * CHUNK
        return [
            pltpu.async_copy(edge_hbm.at[pl.ds(off, CHUNK)], srcs[b], sems[b]),
            pltpu.async_copy(edge_hbm.at[pl.ds(E + off, CHUNK)], dsts[b], sems[b]),
            pltpu.async_copy(w_hbm.at[pl.ds(off, CHUNK)], ws[b], sems[b]),
        ]

    handles = start(0)
    for c in range(NCHUNK):
        nxt = start(c + 1) if c + 1 < NCHUNK else None
        for h in handles:
            h.wait()
        handles = nxt
        b = c % NBUF
        sv, dv, wv = srcs[b], dsts[b], ws[b]

        def body(i, carry):
            o0 = i * (U * L)
            for u in range(U):
                o = o0 + u * L
                s = sv[pl.ds(o, L)]
                d = dv[pl.ds(o, L)]
                w = wv[pl.ds(o, L)]
                m = (s < NIN) & (d >= LO)
                xg = plsc.load_gather(x_v, [s & (NIN - 1)])
                plsc.addupdate_scatter(
                    acc_v, [(d - LO) & (NOUT - 1)], w * xg, mask=m)
            return carry

        lax.fori_loop(0, VECS // U, body, 0)

    pltpu.sync_copy(acc_v, out_hbm.at[pl.ds(wid * NOUT, NOUT)])


def _combine_body(p_ref, b_ref, o_ref):
    o_ref[...] = jnp.tanh(
        jnp.sum(p_ref[...], axis=0, keepdims=True) + b_ref[...]
    )


def kernel(x, edge_index, weights, biases):
    partials = _edge_scan(x, edge_index.reshape(2 * E), weights)
    out = pl.pallas_call(
        _combine_body,
        out_shape=jax.ShapeDtypeStruct((1, NOUT), jnp.float32),
    )(partials.reshape(NW, NOUT), biases[LO:].reshape(1, NOUT))
    return out.reshape(NOUT)


# speedup vs baseline: 215.3220x; 1.3724x over previous
import functools

import jax
import jax.numpy as jnp
from jax import lax
from jax.experimental import pallas as pl
from jax.experimental.pallas import tpu as pltpu
from jax.experimental.pallas import tpu_sc as plsc

N = 100000
E = 1600000
NIN = 512
NOUT = 512
LO = N - NOUT

_info = plsc.get_sparse_core_info()
NC = _info.num_cores
NS = _info.num_subcores
L = _info.num_lanes
NW = NC * NS

E_PER = E // NW
CHUNK = 10000
NCHUNK = E_PER // CHUNK
VECS = CHUNK // L

NBUF = 2
U = 5

assert E_PER * NW == E and NCHUNK * CHUNK == E_PER and VECS * L == CHUNK
assert VECS % U == 0

_mesh = plsc.VectorSubcoreMesh(core_axis_name="c", subcore_axis_name="s")


@functools.partial(
    pl.kernel,
    mesh=_mesh,
    compiler_params=pltpu.CompilerParams(needs_layout_passes=False),
    out_type=jax.ShapeDtypeStruct((NW * NOUT,), jnp.float32),
    scratch_types=[
        pltpu.VMEM((NIN,), jnp.float32),
        pltpu.VMEM((NOUT,), jnp.float32),
        pltpu.VMEM((CHUNK,), jnp.int32),
        pltpu.VMEM((CHUNK,), jnp.int32),
        pltpu.VMEM((CHUNK,), jnp.int32),
        pltpu.VMEM((CHUNK,), jnp.int32),
        pltpu.VMEM((CHUNK,), jnp.float32),
        pltpu.VMEM((CHUNK,), jnp.float32),
        pltpu.SemaphoreType.DMA,
        pltpu.SemaphoreType.DMA,
    ],
)
def _edge_scan(x_hbm, edge_hbm, w_hbm, out_hbm,
               x_v, acc_v, src0, src1, dst0, dst1, w0, w1, sem0, sem1):
    sems = (sem0, sem1)
    srcs = (src0, src1)
    dsts = (dst0, dst1)
    ws = (w0, w1)
    wid = lax.axis_index("s") * NC + lax.axis_index("c")
    base = wid * E_PER

    pltpu.sync_copy(x_hbm, x_v)

    zero = jnp.zeros((L,), jnp.float32)

    def zbody(i, carry):
        acc_v[pl.ds(i * L, L)] = zero
        return carry

    lax.fori_loop(0, NOUT // L, zbody, 0)

    def start(c):
        b = c % NBUF
        off = base + c * CHUNK
        return [
            pltpu.async_copy(edge_hbm.at[pl.ds(off, CHUNK)], srcs[b], sems[b]),
            pltpu.async_copy(edge_hbm.at[pl.ds(E + off, CHUNK)], dsts[b], sems[b]),
            pltpu.async_copy(w_hbm.at[pl.ds(off, CHUNK)], ws[b], sems[b]),
        ]

    handles = start(0)
    for c in range(NCHUNK):
        nxt = start(c + 1) if c + 1 < NCHUNK else None
        for h in handles:
            h.wait()
        handles = nxt
        b = c % NBUF
        sv, dv, wv = srcs[b], dsts[b], ws[b]

        @plsc.parallel_loop(0, CHUNK, L, unroll=U)
        def _(o):
            s = sv[pl.ds(o, L)]
            d = dv[pl.ds(o, L)]
            w = wv[pl.ds(o, L)]
            m = (s < NIN) & (d >= LO)
            xg = plsc.load_gather(x_v, [s & (NIN - 1)])
            plsc.addupdate_scatter(
                acc_v, [(d - LO) & (NOUT - 1)], w * xg, mask=m)

    pltpu.sync_copy(acc_v, out_hbm.at[pl.ds(wid * NOUT, NOUT)])


def _combine_body(p_ref, b_ref, o_ref):
    o_ref[...] = jnp.tanh(
        jnp.sum(p_ref[...], axis=0, keepdims=True) + b_ref[...]
    )


def kernel(x, edge_index, weights, biases):
    partials = _edge_scan(x, edge_index.reshape(2 * E), weights)
    out = pl.pallas_call(
        _combine_body,
        out_shape=jax.ShapeDtypeStruct((1, NOUT), jnp.float32),
    )(partials.reshape(NW, NOUT), biases[LO:].reshape(1, NOUT))
    return out.reshape(NOUT)
